# Initial kernel scaffold; baseline (speedup 1.0000x reference)
import jax
import jax.numpy as jnp
from jax.experimental import pallas as pl


def kernel(samples, block_ids, towers):
    raise NotImplementedError("write your pallas kernel here")


# speedup vs baseline: 4.6077x; 4.6077x over previous
import jax
import jax.numpy as jnp
import numpy as np
from jax import lax
from jax.experimental import pallas as pl
from jax.experimental.pallas import tpu as pltpu
from jax.experimental.pallas import tpu_sc as plsc

NSAMP, NBS, LD = 8, 100000, 16
NB, NK, OD = 4096, 26, 16

NC, NSUB = 2, 16
NW = NC * NSUB
BPW = NB // NW
CB = 8
NCHUNK = BPW // CB
TPC = CB * NK
RPC = NSAMP * TPC
MINOR = 128
NROW = RPC // MINOR
ROWS2 = 2 * NSAMP * NK
R2 = NB * ROWS2


def _pattern():
    i = np.arange(RPC, dtype=np.int64)
    s = i // TPC
    j = i % TPC
    pat = ROWS2 * (j // NK) + 2 * NK * s + 2 * (j % NK)
    return jnp.asarray(pat.reshape(NROW, MINOR), dtype=jnp.int32)


def _body(samples_hbm, ids_hbm, towers_hbm, pat_hbm, out_hbm,
          ids_v, gidx, lidx, tidx, lat_v, tow8_v, sem):
    wid = lax.axis_index("s") * NC + lax.axis_index("c")
    wb0 = wid * BPW

    pltpu.sync_copy(pat_hbm, lidx)
    base = (ROWS2 * wb0).astype(jnp.int32)

    def init_row(r, carry):
        for m in range(MINOR // 16):
            sl = pl.ds(16 * m, 16)
            v = lidx[r, sl] + base
            lidx[r, sl] = v
            tidx[r, sl] = v + 1
        return carry

    lax.fori_loop(0, NROW, init_row, 0)

    def chunk(ci, carry):
        b0 = wb0 + ci * CB
        pltpu.sync_copy(ids_hbm.at[pl.ds(NK * b0, TPC)], ids_v)
        for s in range(NSAMP):
            pltpu.sync_copy(towers_hbm.at[pl.ds(NK * b0, TPC)],
                            tow8_v.at[pl.ds(TPC * s, TPC)])

        for r in range(NROW):
            for m in range(MINOR // 16):
                off = r * MINOR + 16 * m
                s_off = jnp.int32((off // TPC) * NBS)
                gidx[r, pl.ds(16 * m, 16)] = (
                    ids_v[pl.ds(off % TPC, 16)] + s_off)

        def move_row(r, c2):
            sl = pl.ds(r * MINOR, MINOR)
            pltpu.async_copy(samples_hbm.at[gidx.at[r]], lat_v.at[sl],
                             sem).wait()
            pltpu.async_copy(lat_v.at[sl], out_hbm.at[lidx.at[r]],
                             sem).wait()
            pltpu.async_copy(tow8_v.at[sl], out_hbm.at[tidx.at[r]],
                             sem).wait()
            return c2

        lax.fori_loop(0, NROW, move_row, 0)

        def bump_row(r, c2):
            for m in range(MINOR // 16):
                sl = pl.ds(16 * m, 16)
                lidx[r, sl] = lidx[r, sl] + jnp.int32(ROWS2 * CB)
                tidx[r, sl] = tidx[r, sl] + jnp.int32(ROWS2 * CB)
            return c2

        lax.fori_loop(0, NROW, bump_row, 0)
        return carry

    lax.fori_loop(0, NCHUNK, chunk, 0)


def kernel(samples, block_ids, towers):
    samples2d = samples.reshape(NSAMP * NBS, LD)
    ids_flat = block_ids.reshape(NB * NK).astype(jnp.int32)
    towers2d = towers.reshape(NB * NK, OD)

    mesh = plsc.VectorSubcoreMesh(core_axis_name="c", subcore_axis_name="s")
    run = pl.kernel(
        _body,
        out_type=jax.ShapeDtypeStruct((R2, LD), jnp.float32),
        mesh=mesh,
        scratch_types=[
            pltpu.VMEM((TPC,), jnp.int32),
            pltpu.VMEM((NROW, MINOR), jnp.int32),
            pltpu.VMEM((NROW, MINOR), jnp.int32),
            pltpu.VMEM((NROW, MINOR), jnp.int32),
            pltpu.VMEM((RPC, LD), jnp.float32),
            pltpu.VMEM((RPC, OD), jnp.float32),
            pltpu.SemaphoreType.DMA,
        ],
        compiler_params=pltpu.CompilerParams(use_tc_tiling_on_sc=False),
    )
    out2 = run(samples2d, ids_flat, towers2d, _pattern())
    return out2.reshape(NB, NSAMP, NK, 2, LD).reshape(NB, NSAMP, NK, 2 * LD)
